# baseline (device time: 205229 ns/iter reference)
import jax
import jax.numpy as jnp
from jax import lax
from jax.experimental import pallas as pl
from jax.experimental.pallas import tpu as pltpu

N_DEV = 16


def kernel(ids, E):
    v_per, d = E.shape
    t = ids.shape[0]

    my = lax.axis_index("i")
    local = ids - my * v_per
    mask = (local >= 0) & (local < v_per)
    safe = jnp.where(mask, local, 0)
    partial = jnp.where(mask[:, None], E[safe], 0.0).astype(jnp.float32)

    def body(p_ref, out_ref, comm_ref, send_sems, recv_sems):
        my_pos = lax.axis_index("i")
        left = lax.rem(my_pos - 1 + N_DEV, N_DEV)
        right = lax.rem(my_pos + 1, N_DEV)

        barrier_sem = pltpu.get_barrier_semaphore()
        for nbr in (left, right):
            pl.semaphore_signal(
                barrier_sem, inc=1,
                device_id=(nbr,), device_id_type=pl.DeviceIdType.MESH,
            )
        pl.semaphore_wait(barrier_sem, 2)

        out_ref[:, :] = p_ref[:, :]

        for h in range(N_DEV - 1):
            src = p_ref if h == 0 else comm_ref.at[h - 1]
            rdma = pltpu.make_async_remote_copy(
                src_ref=src,
                dst_ref=comm_ref.at[h],
                send_sem=send_sems.at[h],
                recv_sem=recv_sems.at[h],
                device_id=(right,),
                device_id_type=pl.DeviceIdType.MESH,
            )
            rdma.start()
            rdma.wait()
            out_ref[:, :] += comm_ref[h, :, :]

    return pl.pallas_call(
        body,
        out_shape=jax.ShapeDtypeStruct((t, d), jnp.float32),
        in_specs=[pl.BlockSpec(memory_space=pltpu.VMEM)],
        out_specs=pl.BlockSpec(memory_space=pltpu.VMEM),
        scratch_shapes=[
            pltpu.VMEM((N_DEV - 1, t, d), jnp.float32),
            pltpu.SemaphoreType.DMA((N_DEV - 1,)),
            pltpu.SemaphoreType.DMA((N_DEV - 1,)),
        ],
        compiler_params=pltpu.CompilerParams(collective_id=0),
    )(partial)


# device time: 43300 ns/iter; 4.7397x vs baseline; 4.7397x over previous
import jax
import jax.numpy as jnp
from jax import lax
from jax.experimental import pallas as pl
from jax.experimental.pallas import tpu as pltpu

N_DEV = 16


def kernel(ids, E):
    v_per, d = E.shape
    t = ids.shape[0]

    my = lax.axis_index("i")
    local = ids - my * v_per
    mask = (local >= 0) & (local < v_per)
    safe = jnp.where(mask, local, 0)
    partial = jnp.where(mask[:, None], E[safe], 0.0).astype(jnp.float32)

    halves = [t // 2, t // 4, t // 8, t // 16]

    def body(p_ref, out_ref, s0, s1, s2, s3, send_sems, recv_sems):
        idx = lax.axis_index("i")
        z = idx // 4
        p = idx % 4
        x = jnp.where((p == 1) | (p == 2), 1, 0)
        y = jnp.where(p >= 2, 1, 0)

        def to_idx(xx, yy, zz):
            pp = jnp.where(yy == 1, 3 - xx, xx)
            return zz * 4 + pp

        partners = [
            to_idx(1 - x, y, z),
            to_idx(x, 1 - y, z),
            to_idx(x, y, jnp.bitwise_xor(z, 1)),
            to_idx(x, y, jnp.bitwise_xor(z, 2)),
        ]
        bits = [x, y, jnp.bitwise_and(z, 1), z // 2]
        scratches = [s0, s1, s2, s3]

        barrier_sem = pltpu.get_barrier_semaphore()
        for nbr in partners:
            pl.semaphore_signal(
                barrier_sem, inc=1,
                device_id=(nbr,), device_id_type=pl.DeviceIdType.MESH,
            )
        pl.semaphore_wait(barrier_sem, 4)

        out_ref[:, :] = p_ref[:, :]

        off = jnp.int32(0)
        for k in range(4):
            half = halves[k]
            b = bits[k]
            my_off = off + b * half
            pr_off = off + (1 - b) * half
            rdma = pltpu.make_async_remote_copy(
                src_ref=out_ref.at[pl.ds(pr_off, half), :],
                dst_ref=scratches[k],
                send_sem=send_sems.at[k],
                recv_sem=recv_sems.at[k],
                device_id=(partners[k],),
                device_id_type=pl.DeviceIdType.MESH,
            )
            rdma.start()
            rdma.wait()
            out_ref[pl.ds(my_off, half), :] += scratches[k][:, :]
            off = my_off

        for k in reversed(range(4)):
            half = halves[k]
            b = bits[k]
            pr_off = off - b * half + (1 - b) * half
            rdma = pltpu.make_async_remote_copy(
                src_ref=out_ref.at[pl.ds(off, half), :],
                dst_ref=out_ref.at[pl.ds(off, half), :],
                send_sem=send_sems.at[4 + k],
                recv_sem=recv_sems.at[4 + k],
                device_id=(partners[k],),
                device_id_type=pl.DeviceIdType.MESH,
            )
            rdma.start()
            rdma.wait()
            off = off - b * half
            del pr_off

    return pl.pallas_call(
        body,
        out_shape=jax.ShapeDtypeStruct((t, d), jnp.float32),
        in_specs=[pl.BlockSpec(memory_space=pltpu.VMEM)],
        out_specs=pl.BlockSpec(memory_space=pltpu.VMEM),
        scratch_shapes=[
            pltpu.VMEM((halves[0], d), jnp.float32),
            pltpu.VMEM((halves[1], d), jnp.float32),
            pltpu.VMEM((halves[2], d), jnp.float32),
            pltpu.VMEM((halves[3], d), jnp.float32),
            pltpu.SemaphoreType.DMA((8,)),
            pltpu.SemaphoreType.DMA((8,)),
        ],
        compiler_params=pltpu.CompilerParams(collective_id=0),
    )(partial)


# device time: 35469 ns/iter; 5.7862x vs baseline; 1.2208x over previous
import jax
import jax.numpy as jnp
from jax import lax
from jax.experimental import pallas as pl
from jax.experimental.pallas import tpu as pltpu

N_DEV = 16
N_PARTS = 4


def kernel(ids, E):
    v_per, d = E.shape
    t = ids.shape[0]
    rows_per_part = t // N_PARTS

    my = lax.axis_index("i")
    local = ids - my * v_per
    mask = (local >= 0) & (local < v_per)
    safe = jnp.where(mask, local, 0)
    partial = jnp.where(mask[:, None], E[safe], 0.0).astype(jnp.float32)

    halves = [rows_per_part >> (k + 1) for k in range(4)]

    def body(p_ref, out_ref, s0, s1, s2, s3, send_sems, recv_sems):
        idx = lax.axis_index("i")
        z = idx // 4
        p = idx % 4
        x = jnp.where((p == 1) | (p == 2), 1, 0)
        y = jnp.where(p >= 2, 1, 0)

        def to_idx(xx, yy, zz):
            pp = jnp.where(yy == 1, 3 - xx, xx)
            return zz * 4 + pp

        partners = [
            to_idx(1 - x, y, z),
            to_idx(x, 1 - y, z),
            to_idx(x, y, jnp.bitwise_xor(z, 1)),
            to_idx(x, y, jnp.bitwise_xor(z, 2)),
        ]
        bits = [x, y, jnp.bitwise_and(z, 1), z // 2]
        scratches = [s0, s1, s2, s3]

        barrier_sem = pltpu.get_barrier_semaphore()
        for nbr in partners:
            pl.semaphore_signal(
                barrier_sem, inc=1,
                device_id=(nbr,), device_id_type=pl.DeviceIdType.MESH,
            )
        pl.semaphore_wait(barrier_sem, 4)

        out_ref[:, :] = p_ref[:, :]

        def sem_idx(phase, k, a):
            return phase * 16 + k * 4 + a

        off = [jnp.int32(a * rows_per_part) for a in range(N_PARTS)]

        for k in range(4):
            half = halves[k]
            rdmas = []
            for a in range(N_PARTS):
                ax = (a + k) % 4
                b = bits[ax]
                my_off = off[a] + b * half
                pr_off = off[a] + (1 - b) * half
                rdma = pltpu.make_async_remote_copy(
                    src_ref=out_ref.at[pl.ds(pr_off, half), :],
                    dst_ref=scratches[k].at[a],
                    send_sem=send_sems.at[sem_idx(0, k, a)],
                    recv_sem=recv_sems.at[sem_idx(0, k, a)],
                    device_id=(partners[ax],),
                    device_id_type=pl.DeviceIdType.MESH,
                )
                rdma.start()
                rdmas.append((rdma, my_off))
                off[a] = my_off
            for a in range(N_PARTS):
                rdma, my_off = rdmas[a]
                rdma.wait()
                out_ref[pl.ds(my_off, half), :] += scratches[k][a, :, :]

        for k in reversed(range(4)):
            half = halves[k]
            rdmas = []
            for a in range(N_PARTS):
                ax = (a + k) % 4
                b = bits[ax]
                rdma = pltpu.make_async_remote_copy(
                    src_ref=out_ref.at[pl.ds(off[a], half), :],
                    dst_ref=out_ref.at[pl.ds(off[a], half), :],
                    send_sem=send_sems.at[sem_idx(1, k, a)],
                    recv_sem=recv_sems.at[sem_idx(1, k, a)],
                    device_id=(partners[ax],),
                    device_id_type=pl.DeviceIdType.MESH,
                )
                rdma.start()
                rdmas.append(rdma)
                off[a] = off[a] - b * half
            for a in range(N_PARTS):
                rdmas[a].wait()

    return pl.pallas_call(
        body,
        out_shape=jax.ShapeDtypeStruct((t, d), jnp.float32),
        in_specs=[pl.BlockSpec(memory_space=pltpu.VMEM)],
        out_specs=pl.BlockSpec(memory_space=pltpu.VMEM),
        scratch_shapes=[
            pltpu.VMEM((N_PARTS, halves[0], d), jnp.float32),
            pltpu.VMEM((N_PARTS, halves[1], d), jnp.float32),
            pltpu.VMEM((N_PARTS, halves[2], d), jnp.float32),
            pltpu.VMEM((N_PARTS, halves[3], d), jnp.float32),
            pltpu.SemaphoreType.DMA((32,)),
            pltpu.SemaphoreType.DMA((32,)),
        ],
        compiler_params=pltpu.CompilerParams(collective_id=0),
    )(partial)


# device time: 32183 ns/iter; 6.3769x vs baseline; 1.1021x over previous
import jax
import jax.numpy as jnp
from jax import lax
from jax.experimental import pallas as pl
from jax.experimental.pallas import tpu as pltpu

N_DEV = 16
N_PARTS = 4


def kernel(ids, E):
    v_per, d = E.shape
    t = ids.shape[0]
    rows_per_part = t // N_PARTS
    h0 = rows_per_part // 2
    h1 = rows_per_part // 4

    my = lax.axis_index("i")
    local = ids - my * v_per
    mask = (local >= 0) & (local < v_per)
    safe = jnp.where(mask, local, 0)
    partial = jnp.where(mask[:, None], E[safe], 0.0).astype(jnp.float32)

    def body(p_ref, out_ref, s0, s1, s2, s3, send_sems, recv_sems):
        idx = lax.axis_index("i")
        z = idx // 4
        p = idx % 4
        x = jnp.where((p == 1) | (p == 2), 1, 0)
        y = jnp.where(p >= 2, 1, 0)

        def to_idx(xx, yy, zz):
            pp = jnp.where(yy == 1, 3 - xx, xx)
            return zz * 4 + pp

        partners = [
            to_idx(1 - x, y, z),
            to_idx(x, 1 - y, z),
            to_idx(x, y, jnp.bitwise_xor(z, 1)),
            to_idx(x, y, jnp.bitwise_xor(z, 2)),
        ]
        bits = [x, y, jnp.bitwise_and(z, 1), z // 2]

        barrier_sem = pltpu.get_barrier_semaphore()
        for nbr in partners:
            pl.semaphore_signal(
                barrier_sem, inc=1,
                device_id=(nbr,), device_id_type=pl.DeviceIdType.MESH,
            )
        pl.semaphore_wait(barrier_sem, 4)

        out_ref[:, :] = p_ref[:, :]

        def sem_idx(s, a):
            return s * 4 + a

        off = [jnp.int32(a * rows_per_part) for a in range(N_PARTS)]

        def exchange(step, a, src_off, n_rows, axis, dst_scratch):
            if dst_scratch is None:
                dst = out_ref.at[pl.ds(src_off, n_rows), :]
            else:
                dst = dst_scratch
            rdma = pltpu.make_async_remote_copy(
                src_ref=out_ref.at[pl.ds(src_off, n_rows), :],
                dst_ref=dst,
                send_sem=send_sems.at[sem_idx(step, a)],
                recv_sem=recv_sems.at[sem_idx(step, a)],
                device_id=(partners[axis],),
                device_id_type=pl.DeviceIdType.MESH,
            )
            rdma.start()
            return rdma

        rdmas = []
        for a in range(N_PARTS):
            ax = a % 4
            b = bits[ax]
            my_off = off[a] + b * h0
            pr_off = off[a] + (1 - b) * h0
            rdmas.append((exchange(0, a, pr_off, h0, ax, s0.at[a]), my_off))
            off[a] = my_off
        for a in range(N_PARTS):
            rdmas[a][0].wait()
            out_ref[pl.ds(rdmas[a][1], h0), :] += s0[a, :, :]

        rdmas = []
        for a in range(N_PARTS):
            ax = (a + 1) % 4
            b = bits[ax]
            my_off = off[a] + b * h1
            pr_off = off[a] + (1 - b) * h1
            rdmas.append((exchange(1, a, pr_off, h1, ax, s1.at[a]), my_off))
            off[a] = my_off
        for a in range(N_PARTS):
            rdmas[a][0].wait()
            out_ref[pl.ds(rdmas[a][1], h1), :] += s1[a, :, :]

        for step, scratch in ((2, s2), (3, s3)):
            rdmas = []
            for a in range(N_PARTS):
                ax = (a + step) % 4
                rdmas.append(exchange(step, a, off[a], h1, ax, scratch.at[a]))
            for a in range(N_PARTS):
                rdmas[a].wait()
                out_ref[pl.ds(off[a], h1), :] += scratch[a, :, :]

        rdmas = []
        for a in range(N_PARTS):
            ax = (a + 1) % 4
            rdmas.append(exchange(4, a, off[a], h1, ax, None))
            off[a] = off[a] - bits[ax] * h1
        for a in range(N_PARTS):
            rdmas[a].wait()

        rdmas = []
        for a in range(N_PARTS):
            ax = a % 4
            rdmas.append(exchange(5, a, off[a], h0, ax, None))
            off[a] = off[a] - bits[ax] * h0
        for a in range(N_PARTS):
            rdmas[a].wait()

    return pl.pallas_call(
        body,
        out_shape=jax.ShapeDtypeStruct((t, d), jnp.float32),
        in_specs=[pl.BlockSpec(memory_space=pltpu.VMEM)],
        out_specs=pl.BlockSpec(memory_space=pltpu.VMEM),
        scratch_shapes=[
            pltpu.VMEM((N_PARTS, h0, d), jnp.float32),
            pltpu.VMEM((N_PARTS, h1, d), jnp.float32),
            pltpu.VMEM((N_PARTS, h1, d), jnp.float32),
            pltpu.VMEM((N_PARTS, h1, d), jnp.float32),
            pltpu.SemaphoreType.DMA((24,)),
            pltpu.SemaphoreType.DMA((24,)),
        ],
        compiler_params=pltpu.CompilerParams(collective_id=0),
    )(partial)


# device time: 28073 ns/iter; 7.3105x vs baseline; 1.1464x over previous
import jax
import jax.numpy as jnp
from jax import lax
from jax.experimental import pallas as pl
from jax.experimental.pallas import tpu as pltpu

N_DEV = 16
N_PARTS = 4
N_STEPS = 6


def kernel(ids, E):
    v_per, d = E.shape
    t = ids.shape[0]
    rows_per_part = t // N_PARTS
    h0 = rows_per_part // 2
    h1 = rows_per_part // 4

    my = lax.axis_index("i")
    local = ids - my * v_per
    mask = (local >= 0) & (local < v_per)
    safe = jnp.where(mask, local, 0)
    partial = jnp.where(mask[:, None], E[safe], 0.0).astype(jnp.float32)

    def body(p_ref, out_ref, s0, s1, s2, s3, send_sems, recv_sems):
        idx = lax.axis_index("i")
        z = idx // 4
        p = idx % 4
        x = jnp.where((p == 1) | (p == 2), 1, 0)
        y = jnp.where(p >= 2, 1, 0)

        def to_idx(xx, yy, zz):
            pp = jnp.where(yy == 1, 3 - xx, xx)
            return zz * 4 + pp

        partners = [
            to_idx(1 - x, y, z),
            to_idx(x, 1 - y, z),
            to_idx(x, y, jnp.bitwise_xor(z, 1)),
            to_idx(x, y, jnp.bitwise_xor(z, 2)),
        ]
        bits = [x, y, jnp.bitwise_and(z, 1), z // 2]
        scratches = [s0, s1, s2, s3]

        barrier_sem = pltpu.get_barrier_semaphore()
        for nbr in partners:
            pl.semaphore_signal(
                barrier_sem, inc=1,
                device_id=(nbr,), device_id_type=pl.DeviceIdType.MESH,
            )
        pl.semaphore_wait(barrier_sem, 4)

        def sem_idx(s, a):
            return s * N_PARTS + a

        off = [jnp.int32(a * rows_per_part) for a in range(N_PARTS)]

        def start(step, a):
            if step == 0 or step == 1:
                half = h0 if step == 0 else h1
                ax = (a + step) % 4
                b = bits[ax]
                my_off = off[a] + b * half
                pr_off = off[a] + (1 - b) * half
                src = out_ref.at[pl.ds(pr_off, half), :]
                dst = scratches[step].at[a]
                off[a] = my_off
                add_off = my_off
            elif step == 2 or step == 3:
                ax = (a + step) % 4
                src = out_ref.at[pl.ds(off[a], h1), :]
                dst = scratches[step].at[a]
                add_off = off[a]
            else:
                slot = 5 - step
                half = h1 if slot == 1 else h0
                ax = (a + slot) % 4
                src = out_ref.at[pl.ds(off[a], half), :]
                dst = out_ref.at[pl.ds(off[a], half), :]
                off[a] = off[a] - bits[ax] * half
                add_off = None
            rdma = pltpu.make_async_remote_copy(
                src_ref=src,
                dst_ref=dst,
                send_sem=send_sems.at[sem_idx(step, a)],
                recv_sem=recv_sems.at[sem_idx(step, a)],
                device_id=(partners[ax],),
                device_id_type=pl.DeviceIdType.MESH,
            )
            rdma.start()
            return rdma, add_off

        n_rows = [h0, h1, h1, h1, h1, h0]
        inflight = [start(0, a) for a in range(N_PARTS)]
        for step in range(1, N_STEPS):
            for a in range(N_PARTS):
                rdma, add_off = inflight[a]
                rdma.wait()
                if add_off is not None:
                    out_ref[pl.ds(add_off, n_rows[step - 1]), :] += (
                        scratches[step - 1][a, :, :]
                    )
                inflight[a] = start(step, a)
        for a in range(N_PARTS):
            inflight[a][0].wait()

    return pl.pallas_call(
        body,
        out_shape=jax.ShapeDtypeStruct((t, d), jnp.float32),
        in_specs=[pl.BlockSpec(memory_space=pltpu.VMEM)],
        out_specs=pl.BlockSpec(memory_space=pltpu.VMEM),
        input_output_aliases={0: 0},
        scratch_shapes=[
            pltpu.VMEM((N_PARTS, h0, d), jnp.float32),
            pltpu.VMEM((N_PARTS, h1, d), jnp.float32),
            pltpu.VMEM((N_PARTS, h1, d), jnp.float32),
            pltpu.VMEM((N_PARTS, h1, d), jnp.float32),
            pltpu.SemaphoreType.DMA((N_STEPS * N_PARTS,)),
            pltpu.SemaphoreType.DMA((N_STEPS * N_PARTS,)),
        ],
        compiler_params=pltpu.CompilerParams(collective_id=0),
    )(partial)
